# baseline (speedup 1.0000x reference)
import functools

import jax
import jax.numpy as jnp
from jax import lax
from jax.experimental import pallas as pl
from jax.experimental.pallas import tpu as pltpu
from jax.experimental.pallas import tpu_sc as plsc

H = 256
W = 256
B = 8
N = 65536

NC = 2
NS = 16
LANES = 16

BPC = B // NC
PPC = BPC * N
PPT = PPC // NS
CH = 8192
NCH = PPT // CH
VECS = CH // LANES
ROWS = CH // 128

GBINS = BPC * H * W
GSZ = GBINS + 128


def _project_scatter(xs, ys, zs, dens):
    mesh = plsc.VectorSubcoreMesh(
        core_axis_name="c", subcore_axis_name="s",
        num_cores=NC, num_subcores=NS)

    @functools.partial(
        pl.kernel,
        out_type=jax.ShapeDtypeStruct((B * N,), jnp.float32),
        mesh=mesh,
        compiler_params=pltpu.CompilerParams(needs_layout_passes=False),
        scratch_types=[
            pltpu.VMEM((NCH, CH), jnp.float32),
            pltpu.VMEM((NCH, CH), jnp.float32),
            pltpu.VMEM((NCH, CH), jnp.float32),
            pltpu.VMEM((NCH, CH), jnp.float32),
            pltpu.VMEM((NCH * ROWS, 128), jnp.int32),
            pltpu.VMEM((NCH * ROWS, 128), jnp.float32),
            pltpu.VMEM((CH, ), jnp.float32),
            pltpu.VMEM_SHARED((GSZ,), jnp.float32),
            [pltpu.SemaphoreType.DMA] * NCH,
            pltpu.SemaphoreType.DMA,
        ],
    )
    def k(xs_hbm, ys_hbm, zs_hbm, dens_hbm, out_hbm,
          xs_v, ys_v, zs_v_, dens_v, idx_v, upd_v, zbuf_v,
          grid_sh, sem_in, sem_sc):
        cid = lax.axis_index("c")
        sid = lax.axis_index("s")

        b = cid * BPC + sid // (NS // BPC)
        boff = (sid // (NS // BPC)) * (H * W)
        p0 = (sid % (NS // BPC)) * PPT
        dummy = GBINS + sid * 8

        in_handles = []
        for ch in range(NCH):
            g0 = p0 + ch * CH
            in_handles.append([
                pltpu.async_copy(src.at[b, pl.ds(g0, CH)], dst.at[ch],
                                 sem_in[ch])
                for src, dst in ((xs_hbm, xs_v), (ys_hbm, ys_v),
                                 (zs_hbm, zs_v_), (dens_hbm, dens_v))
            ])

        zero16 = jnp.zeros((LANES,), jnp.float32)

        @pl.loop(0, CH, step=LANES)
        def _(i):
            zbuf_v[pl.ds(i, LANES)] = zero16

        for t in range(PPT // CH):
            pltpu.sync_copy(zbuf_v, grid_sh.at[pl.ds(sid * PPT + t * CH, CH)])

        @pl.when(sid == 0)
        def _():
            pltpu.sync_copy(zbuf_v.at[pl.ds(0, 128)],
                            grid_sh.at[pl.ds(GBINS, 128)])

        plsc.subcore_barrier()

        for ch in range(NCH):
            for h in in_handles[ch]:
                h.wait()

            @pl.loop(0, ROWS)
            def _(j):
                gj = ch * ROWS + j

                @pl.loop(0, 8)
                def _(kk):
                    i = j * 8 + kk
                    x = xs_v[ch, pl.ds(i * LANES, LANES)]
                    y = ys_v[ch, pl.ds(i * LANES, LANES)]
                    z = zs_v_[ch, pl.ds(i * LANES, LANES)]
                    d = dens_v[ch, pl.ds(i * LANES, LANES)]
                    r = 1.0 / z
                    u = x * r + 0.5
                    v = y * r + 0.5
                    uf = jnp.minimum(jnp.maximum(u * float(W), 0.0),
                                     float(W - 1))
                    vf = jnp.minimum(jnp.maximum(v * float(H), 0.0),
                                     float(H - 1))
                    ui = uf.astype(jnp.int32)
                    vi = vf.astype(jnp.int32)
                    keep = d > 0.5
                    contrib = jnp.where(keep, z * d, 0.0)
                    bin_ = jnp.where(keep, boff + vi * W + ui, dummy)
                    idx_v[gj, pl.ds(kk * LANES, LANES)] = bin_
                    upd_v[gj, pl.ds(kk * LANES, LANES)] = contrib

                pltpu.async_copy(upd_v.at[gj], grid_sh.at[idx_v.at[gj]],
                                 sem_sc, add=True)

        @pl.loop(0, NCH * ROWS)
        def _(j):
            pltpu.make_async_copy(out_hbm.at[pl.ds(0, 128)], upd_v.at[0],
                                  sem_sc).wait()

        plsc.subcore_barrier()

        pltpu.sync_copy(grid_sh.at[pl.ds(sid * PPT, PPT)],
                        out_hbm.at[pl.ds(cid * PPC + sid * PPT, PPT)])

    return k(xs, ys, zs, dens)


def _conv_body(imgp_ref, w_ref, b_ref, out_ref):
    acc = jnp.zeros((H, W), jnp.float32)
    for c in range(3):
        for dh in range(3):
            for dw in range(3):
                acc = acc + (w_ref[c * 9 + dh * 3 + dw] *
                             imgp_ref[0, c, dh:dh + H, dw:dw + W])
    out_ref[0] = jnp.maximum(acc + b_ref[0], 0.0)


def _conv(imgp, w_flat, b_conv):
    return pl.pallas_call(
        _conv_body,
        grid=(B,),
        in_specs=[
            pl.BlockSpec((1, 3, H + 2, W + 2), lambda b: (b, 0, 0, 0)),
            pl.BlockSpec(memory_space=pltpu.SMEM),
            pl.BlockSpec(memory_space=pltpu.SMEM),
        ],
        out_specs=pl.BlockSpec((1, H, W), lambda b: (b, 0, 0)),
        out_shape=jax.ShapeDtypeStruct((B, H, W), jnp.float32),
    )(imgp, w_flat, b_conv)


def _loss_body(proj_ref, depth_ref, out_ref):
    p = proj_ref[...]
    dpt = depth_ref[...]
    m = (p > 0.0).astype(jnp.float32)
    diff = p - dpt
    ssq = jnp.sum(diff * diff * m)
    sm = jnp.sum(m)
    out_ref[0, 0] = ssq / jnp.maximum(sm, 1.0)


def _loss(proj2d, depth2d):
    return pl.pallas_call(
        _loss_body,
        out_specs=pl.BlockSpec(memory_space=pltpu.SMEM),
        out_shape=jax.ShapeDtypeStruct((1, 1), jnp.float32),
    )(proj2d, depth2d)


def kernel(images, points, densities, W_conv, b_conv):
    imgp = jnp.pad(images, ((0, 0), (0, 0), (1, 1), (1, 1)))
    w_flat = W_conv.reshape(27)
    depth = _conv(imgp, w_flat, b_conv)
    proj = _project_scatter(points[:, :, 0], points[:, :, 1],
                            points[:, :, 2], densities[:, :, 0])
    loss = _loss(proj.reshape(512, 1024), depth.reshape(512, 1024))
    return loss[0, 0]


# speedup vs baseline: 1.0071x; 1.0071x over previous
import functools

import jax
import jax.numpy as jnp
from jax import lax
from jax.experimental import pallas as pl
from jax.experimental.pallas import tpu as pltpu
from jax.experimental.pallas import tpu_sc as plsc

H = 256
W = 256
B = 8
N = 65536

NC = 2
NS = 16
LANES = 16

BPC = B // NC
PPC = BPC * N
PPT = PPC // NS
CH = 4096
NCH = PPT // CH
VECS = CH // LANES
ROWS = CH // 128

GBINS = BPC * H * W
GSZ = GBINS + 128


def _project_scatter(xs, ys, zs, dens):
    mesh = plsc.VectorSubcoreMesh(
        core_axis_name="c", subcore_axis_name="s",
        num_cores=NC, num_subcores=NS)

    @functools.partial(
        pl.kernel,
        out_type=jax.ShapeDtypeStruct((B * N,), jnp.float32),
        mesh=mesh,
        compiler_params=pltpu.CompilerParams(needs_layout_passes=False),
        scratch_types=[
            pltpu.VMEM((NCH, CH), jnp.float32),
            pltpu.VMEM((NCH, CH), jnp.float32),
            pltpu.VMEM((NCH, CH), jnp.float32),
            pltpu.VMEM((NCH, CH), jnp.float32),
            pltpu.VMEM((NCH * ROWS, 128), jnp.int32),
            pltpu.VMEM((NCH * ROWS, 128), jnp.float32),
            pltpu.VMEM((CH, ), jnp.float32),
            pltpu.VMEM_SHARED((GSZ,), jnp.float32),
            [pltpu.SemaphoreType.DMA] * NCH,
            pltpu.SemaphoreType.DMA,
        ],
    )
    def k(xs_hbm, ys_hbm, zs_hbm, dens_hbm, out_hbm,
          xs_v, ys_v, zs_v_, dens_v, idx_v, upd_v, zbuf_v,
          grid_sh, sem_in, sem_sc):
        cid = lax.axis_index("c")
        sid = lax.axis_index("s")

        b = cid * BPC + sid // (NS // BPC)
        boff = (sid // (NS // BPC)) * (H * W)
        p0 = (sid % (NS // BPC)) * PPT
        dummy = GBINS + sid * 8

        in_handles = []
        for ch in range(NCH):
            g0 = p0 + ch * CH
            in_handles.append([
                pltpu.async_copy(src.at[b, pl.ds(g0, CH)], dst.at[ch],
                                 sem_in[ch])
                for src, dst in ((xs_hbm, xs_v), (ys_hbm, ys_v),
                                 (zs_hbm, zs_v_), (dens_hbm, dens_v))
            ])

        zero16 = jnp.zeros((LANES,), jnp.float32)

        @pl.loop(0, CH, step=LANES)
        def _(i):
            zbuf_v[pl.ds(i, LANES)] = zero16

        for t in range(PPT // CH):
            pltpu.sync_copy(zbuf_v, grid_sh.at[pl.ds(sid * PPT + t * CH, CH)])

        @pl.when(sid == 0)
        def _():
            pltpu.sync_copy(zbuf_v.at[pl.ds(0, 128)],
                            grid_sh.at[pl.ds(GBINS, 128)])

        plsc.subcore_barrier()

        for ch in range(NCH):
            for h in in_handles[ch]:
                h.wait()

            @pl.loop(0, ROWS)
            def _(j):
                gj = ch * ROWS + j

                @pl.loop(0, 8)
                def _(kk):
                    i = j * 8 + kk
                    x = xs_v[ch, pl.ds(i * LANES, LANES)]
                    y = ys_v[ch, pl.ds(i * LANES, LANES)]
                    z = zs_v_[ch, pl.ds(i * LANES, LANES)]
                    d = dens_v[ch, pl.ds(i * LANES, LANES)]
                    r = 1.0 / z
                    u = x * r + 0.5
                    v = y * r + 0.5
                    uf = jnp.minimum(jnp.maximum(u * float(W), 0.0),
                                     float(W - 1))
                    vf = jnp.minimum(jnp.maximum(v * float(H), 0.0),
                                     float(H - 1))
                    ui = uf.astype(jnp.int32)
                    vi = vf.astype(jnp.int32)
                    keep = d > 0.5
                    contrib = jnp.where(keep, z * d, 0.0)
                    bin_ = jnp.where(keep, boff + vi * W + ui, dummy)
                    idx_v[gj, pl.ds(kk * LANES, LANES)] = bin_
                    upd_v[gj, pl.ds(kk * LANES, LANES)] = contrib

                pltpu.async_copy(upd_v.at[gj], grid_sh.at[idx_v.at[gj]],
                                 sem_sc, add=True)

        @pl.loop(0, NCH * ROWS)
        def _(j):
            pltpu.make_async_copy(out_hbm.at[pl.ds(0, 128)], upd_v.at[0],
                                  sem_sc).wait()

        plsc.subcore_barrier()

        pltpu.sync_copy(grid_sh.at[pl.ds(sid * PPT, PPT)],
                        out_hbm.at[pl.ds(cid * PPC + sid * PPT, PPT)])

    return k(xs, ys, zs, dens)


def _conv_body(imgp_ref, w_ref, b_ref, out_ref):
    acc = jnp.zeros((H, W), jnp.float32)
    for c in range(3):
        for dh in range(3):
            for dw in range(3):
                acc = acc + (w_ref[c * 9 + dh * 3 + dw] *
                             imgp_ref[0, c, dh:dh + H, dw:dw + W])
    out_ref[0] = jnp.maximum(acc + b_ref[0], 0.0)


def _conv(imgp, w_flat, b_conv):
    return pl.pallas_call(
        _conv_body,
        grid=(B,),
        in_specs=[
            pl.BlockSpec((1, 3, H + 2, W + 2), lambda b: (b, 0, 0, 0)),
            pl.BlockSpec(memory_space=pltpu.SMEM),
            pl.BlockSpec(memory_space=pltpu.SMEM),
        ],
        out_specs=pl.BlockSpec((1, H, W), lambda b: (b, 0, 0)),
        out_shape=jax.ShapeDtypeStruct((B, H, W), jnp.float32),
    )(imgp, w_flat, b_conv)


def _loss_body(proj_ref, depth_ref, out_ref):
    p = proj_ref[...]
    dpt = depth_ref[...]
    m = (p > 0.0).astype(jnp.float32)
    diff = p - dpt
    ssq = jnp.sum(diff * diff * m)
    sm = jnp.sum(m)
    out_ref[0, 0] = ssq / jnp.maximum(sm, 1.0)


def _loss(proj2d, depth2d):
    return pl.pallas_call(
        _loss_body,
        out_specs=pl.BlockSpec(memory_space=pltpu.SMEM),
        out_shape=jax.ShapeDtypeStruct((1, 1), jnp.float32),
    )(proj2d, depth2d)


def kernel(images, points, densities, W_conv, b_conv):
    imgp = jnp.pad(images, ((0, 0), (0, 0), (1, 1), (1, 1)))
    w_flat = W_conv.reshape(27)
    depth = _conv(imgp, w_flat, b_conv)
    proj = _project_scatter(points[:, :, 0], points[:, :, 1],
                            points[:, :, 2], densities[:, :, 0])
    loss = _loss(proj.reshape(512, 1024), depth.reshape(512, 1024))
    return loss[0, 0]
